# Initial kernel scaffold; baseline (speedup 1.0000x reference)
import jax
import jax.numpy as jnp
from jax.experimental import pallas as pl


def kernel(h, t_ij, edge_index, Wq, bq, Wk, bk, W1, b1, W2, b2, Wre, bre, Wc, bc):
    raise NotImplementedError("write your pallas kernel here")


# speedup vs baseline: 7.1581x; 7.1581x over previous
import jax
import jax.numpy as jnp
from jax import lax
from jax.experimental import pallas as pl
from jax.experimental.pallas import tpu as pltpu
from jax.experimental.pallas import tpu_sc as plsc

N = 10000
E = 320000
D = 128
NPAD = 10240
NC, NS = 2, 16
NW = NC * NS
EW = E // NW
C = 80
NCHUNK = EW // C

BN = 2000
BE = 4000


def _proj_body(h_ref, wq_ref, wk_ref, w1_ref, w2_ref,
               bq_ref, bk_ref, b1_ref, b2_ref, q_ref, k_ref, v_ref):
    hb = h_ref[...]
    q_ref[...] = jnp.dot(hb, wq_ref[...], preferred_element_type=jnp.float32) + bq_ref[...]
    k_ref[...] = jnp.dot(hb, wk_ref[...], preferred_element_type=jnp.float32) + bk_ref[...]
    vh = jnp.maximum(jnp.dot(hb, w1_ref[...], preferred_element_type=jnp.float32) + b1_ref[...], 0.0)
    v_ref[...] = jnp.dot(vh, w2_ref[...], preferred_element_type=jnp.float32) + b2_ref[...]


def _tc_proj(h, WqT, WkT, W1T, W2T, bq, bk, b1, b2):
    nblk = N // BN
    w_spec = pl.BlockSpec((D, D), lambda i: (0, 0))
    b_spec = pl.BlockSpec((1, D), lambda i: (0, 0))
    r_spec = pl.BlockSpec((BN, D), lambda i: (i, 0))
    return pl.pallas_call(
        _proj_body,
        grid=(nblk,),
        in_specs=[r_spec, w_spec, w_spec, w_spec, w_spec,
                  b_spec, b_spec, b_spec, b_spec],
        out_specs=[r_spec, r_spec, r_spec],
        out_shape=[jax.ShapeDtypeStruct((N, D), jnp.float32)] * 3,
    )(h, WqT, WkT, W1T, W2T, bq, bk, b1, b2)


def _gather3_body(q_hbm, k_hbm, v_hbm, ni_hbm, nj_hbm,
                  qg_hbm, kg_hbm, vg_hbm, idx_i, idx_j, bq, bk, bv, sem):
    wid = lax.axis_index("s") * NC + lax.axis_index("c")

    def chunk(c, _):
        base = wid * EW + c * C
        pltpu.sync_copy(ni_hbm.at[pl.ds(base, C)], idx_i)
        pltpu.sync_copy(nj_hbm.at[pl.ds(base, C)], idx_j)
        c1 = pltpu.async_copy(q_hbm.at[idx_i], bq, sem)
        c2 = pltpu.async_copy(k_hbm.at[idx_j], bk, sem)
        c3 = pltpu.async_copy(v_hbm.at[idx_j], bv, sem)
        c1.wait()
        c2.wait()
        c3.wait()
        pltpu.sync_copy(bq, qg_hbm.at[pl.ds(base, C)])
        pltpu.sync_copy(bk, kg_hbm.at[pl.ds(base, C)])
        pltpu.sync_copy(bv, vg_hbm.at[pl.ds(base, C)])
        return _

    lax.fori_loop(0, NCHUNK, chunk, 0)


def _sc_gather3(q, k, v, n_i, n_j):
    mesh = plsc.VectorSubcoreMesh(core_axis_name="c", subcore_axis_name="s")
    f = pl.kernel(
        _gather3_body,
        out_type=[jax.ShapeDtypeStruct((E, D), jnp.float32)] * 3,
        mesh=mesh,
        scratch_types=[
            pltpu.VMEM((C,), jnp.int32),
            pltpu.VMEM((C,), jnp.int32),
            pltpu.VMEM((C, D), jnp.float32),
            pltpu.VMEM((C, D), jnp.float32),
            pltpu.VMEM((C, D), jnp.float32),
            pltpu.SemaphoreType.DMA,
        ],
    )
    return f(q, k, v, n_i, n_j)


def _ex_body(qg_ref, kg_ref, t_ref, wre_ref, bre_ref, out_ref):
    re = jnp.maximum(
        jnp.dot(t_ref[...], wre_ref[...], preferred_element_type=jnp.float32)
        + bre_ref[...], 0.0)
    prod = qg_ref[...] * kg_ref[...] * re
    row = lax.broadcasted_iota(jnp.int32, (D, 16), 0)
    col = lax.broadcasted_iota(jnp.int32, (D, 16), 1)
    sel = jnp.where((row // 16) == col, 1.0, 0.0).astype(jnp.float32)
    a16 = jnp.dot(prod, sel, preferred_element_type=jnp.float32)
    lane = lax.broadcasted_iota(jnp.int32, a16.shape, 1)
    out_ref[...] = jnp.where(lane == 8, 1.0,
                             jnp.where(lane < 8, jnp.exp(a16), 0.0))


def _tc_ex(qg, kg, t_ij, WreT, bre):
    nblk = E // BE
    return pl.pallas_call(
        _ex_body,
        grid=(nblk,),
        in_specs=[pl.BlockSpec((BE, D), lambda i: (i, 0)),
                  pl.BlockSpec((BE, D), lambda i: (i, 0)),
                  pl.BlockSpec((BE, 16), lambda i: (i, 0)),
                  pl.BlockSpec((16, D), lambda i: (0, 0)),
                  pl.BlockSpec((1, D), lambda i: (0, 0))],
        out_specs=pl.BlockSpec((BE, 16), lambda i: (i, 0)),
        out_shape=jax.ShapeDtypeStruct((E, 16), jnp.float32),
    )(qg, kg, t_ij, WreT, bre)


ROWS_PER_TILE = NPAD // NS


def _scatter_body(ex_hbm, ni_hbm, zero_hbm, acc2_hbm, idx, exbuf, acc_sp, sem):
    cid = lax.axis_index("c")
    sid = lax.axis_index("s")
    wid = sid * NC + cid
    pltpu.sync_copy(zero_hbm.at[pl.ds(sid * ROWS_PER_TILE, ROWS_PER_TILE)],
                    acc_sp.at[pl.ds(sid * ROWS_PER_TILE, ROWS_PER_TILE)])
    plsc.subcore_barrier()

    def chunk(c, _):
        base = wid * EW + c * C
        pltpu.sync_copy(ni_hbm.at[pl.ds(base, C)], idx)
        pltpu.sync_copy(ex_hbm.at[pl.ds(base, C)], exbuf)
        pltpu.sync_copy(exbuf, acc_sp.at[idx], add=True)
        return _

    lax.fori_loop(0, NCHUNK, chunk, 0)
    plsc.subcore_barrier()
    pltpu.sync_copy(acc_sp.at[pl.ds(sid * ROWS_PER_TILE, ROWS_PER_TILE)],
                    acc2_hbm.at[cid, pl.ds(sid * ROWS_PER_TILE, ROWS_PER_TILE)])


def _sc_scatter(exrow, n_i, zero):
    mesh = plsc.VectorSubcoreMesh(core_axis_name="c", subcore_axis_name="s")
    f = pl.kernel(
        _scatter_body,
        out_type=jax.ShapeDtypeStruct((NC, NPAD, 16), jnp.float32),
        mesh=mesh,
        scratch_types=[
            pltpu.VMEM((C,), jnp.int32),
            pltpu.VMEM((C, 16), jnp.float32),
            pltpu.VMEM_SHARED((NPAD, 16), jnp.float32),
            pltpu.SemaphoreType.DMA,
        ],
    )
    return f(exrow, n_i, zero)


def _scale_body(a0_ref, a1_ref, out_ref):
    acc = a0_ref[...] + a1_ref[...]
    safe = jnp.where(acc == 0.0, 1.0, acc)
    sq = jnp.sqrt(acc[:, 8:9])
    scale16 = sq / (4.0 * safe)
    lane = lax.broadcasted_iota(jnp.int32, (NPAD, D), 1)
    row = lax.broadcasted_iota(jnp.int32, (16, D), 0)
    col = lax.broadcasted_iota(jnp.int32, (16, D), 1)
    put = jnp.where((col == row) & (col < 8), 1.0, 0.0).astype(jnp.float32)
    out_ref[...] = jnp.dot(scale16, put, preferred_element_type=jnp.float32)


def _tc_scale(acc2):
    return pl.pallas_call(
        _scale_body,
        grid=(1,),
        in_specs=[pl.BlockSpec((NPAD, 16), lambda i: (0, 0)),
                  pl.BlockSpec((NPAD, 16), lambda i: (0, 0))],
        out_specs=pl.BlockSpec((NPAD, D), lambda i: (0, 0)),
        out_shape=jax.ShapeDtypeStruct((NPAD, D), jnp.float32),
    )(acc2[0], acc2[1])


def _coef_body(ex_hbm, ni_hbm, scale_hbm, coef_hbm, idx, exbuf, sbuf, sem):
    wid = lax.axis_index("s") * NC + lax.axis_index("c")

    def chunk(c, _):
        base = wid * EW + c * C
        pltpu.sync_copy(ni_hbm.at[pl.ds(base, C)], idx)
        pltpu.sync_copy(ex_hbm.at[pl.ds(base, C)], exbuf)
        pltpu.async_copy(scale_hbm.at[idx], sbuf, sem).wait()

        def row(i, _):
            exbuf[i, :] = exbuf[i, :] * sbuf[i, pl.ds(0, 16)]
            return _

        lax.fori_loop(0, C, row, 0)
        pltpu.sync_copy(exbuf, coef_hbm.at[pl.ds(base, C)])
        return _

    lax.fori_loop(0, NCHUNK, chunk, 0)


def _sc_coef(exrow, n_i, scale):
    mesh = plsc.VectorSubcoreMesh(core_axis_name="c", subcore_axis_name="s")
    f = pl.kernel(
        _coef_body,
        out_type=jax.ShapeDtypeStruct((E, 16), jnp.float32),
        mesh=mesh,
        scratch_types=[
            pltpu.VMEM((C,), jnp.int32),
            pltpu.VMEM((C, 16), jnp.float32),
            pltpu.VMEM((C, D), jnp.float32),
            pltpu.SemaphoreType.DMA,
        ],
    )
    return f(exrow, n_i, scale)


def _comb_body(coef_ref, vg_ref, wc_ref, bc_ref, out_ref):
    row = lax.broadcasted_iota(jnp.int32, (16, D), 0)
    col = lax.broadcasted_iota(jnp.int32, (16, D), 1)
    expand = jnp.where((col // 16) == row, 1.0, 0.0).astype(jnp.float32)
    c128 = jnp.dot(coef_ref[...], expand, preferred_element_type=jnp.float32)
    out_ref[...] = jnp.dot(c128 * vg_ref[...], wc_ref[...],
                           preferred_element_type=jnp.float32) + bc_ref[...]


def _tc_comb(coef, vg, WcT, bc):
    nblk = E // BE
    return pl.pallas_call(
        _comb_body,
        grid=(nblk,),
        in_specs=[pl.BlockSpec((BE, 16), lambda i: (i, 0)),
                  pl.BlockSpec((BE, D), lambda i: (i, 0)),
                  pl.BlockSpec((D, D), lambda i: (0, 0)),
                  pl.BlockSpec((1, D), lambda i: (0, 0))],
        out_specs=pl.BlockSpec((BE, D), lambda i: (i, 0)),
        out_shape=jax.ShapeDtypeStruct((E, D), jnp.float32),
    )(coef, vg, WcT, bc)


def kernel(h, t_ij, edge_index, Wq, bq, Wk, bk, W1, b1, W2, b2, Wre, bre, Wc, bc):
    n_j = edge_index[0]
    n_i = edge_index[1]
    q, k, v = _tc_proj(h, Wq.T, Wk.T, W1.T, W2.T,
                       bq.reshape(1, D), bk.reshape(1, D),
                       b1.reshape(1, D), b2.reshape(1, D))
    qg, kg, vg = _sc_gather3(q, k, v, n_i, n_j)
    exrow = _tc_ex(qg, kg, t_ij, Wre.T, bre.reshape(1, D))
    zero = jnp.zeros((NPAD, 16), jnp.float32)
    acc2 = _sc_scatter(exrow, n_i, zero)
    scale = _tc_scale(acc2)
    coef = _sc_coef(exrow, n_i, scale)
    return _tc_comb(coef, vg, Wc.T, bc.reshape(1, D))
